# baseline (speedup 1.0000x reference)
import functools
import jax
import jax.numpy as jnp
from jax import lax
from jax.experimental import pallas as pl
from jax.experimental.pallas import tpu as pltpu
from jax.experimental.pallas import tpu_sc as plsc

NC = 2
NS = 16
NW = NC * NS
LANES = 16

B = 16384
L = 50
D = 64
V = 1000000

G = 128
BPW = B // NW
QPW = BPW // G
UNITS = L * QPW
NBLK = B // G
DBLK = D // 8
NGRP = G // LANES
PITCH = 137

WBLK = 256
WGRID = (V + WBLK - 1) // WBLK


def _wprep_body(i_ref, o_ref):
    t = i_ref[...].T
    o_ref[...] = jnp.concatenate(
        [t, jnp.zeros((WBLK, D), jnp.float32)], axis=1)


_wprep = functools.partial(
    pl.pallas_call,
    grid=(WGRID,),
    in_specs=[pl.BlockSpec((D, WBLK), lambda g: (0, g))],
    out_specs=pl.BlockSpec((WBLK, 2 * D), lambda g: (g, 0)),
    out_shape=jax.ShapeDtypeStruct((V, 2 * D), jnp.float32),
)(_wprep_body)


def _embed_body(x_hbm, xlen_hbm, w_hbm, out5_hbm, ret4_hbm,
                idx0, idx1, pairs0, pairs1, stage0, stage1,
                acc_v, inv_v,
                sem_g0, sem_g1, sem_o0, sem_o1):
    idx = (idx0, idx1)
    pairs = (pairs0, pairs1)
    stage = (stage0, stage1)
    sem_g = (sem_g0, sem_g1)
    sem_o = (sem_o0, sem_o1)

    wid = lax.axis_index("s") * NC + lax.axis_index("c")
    b0 = wid * BPW

    pltpu.sync_copy(xlen_hbm.at[pl.ds(pl.multiple_of(b0, BPW), BPW)], inv_v)
    for k in range(BPW // LANES):
        inv_v[pl.ds(k * LANES, LANES)] = 1.0 / inv_v[pl.ds(k * LANES, LANES)]

    zero = jnp.zeros((LANES,), jnp.float32)

    def zero_body(dblk, carry):
        for q in range(QPW):
            for sub in range(8):
                for g in range(NGRP):
                    acc_v[dblk, q, sub, pl.ds(g * LANES, LANES)] = zero
        return carry

    lax.fori_loop(0, DBLK, zero_body, 0)

    def stage_idx(u, b):
        l = u // QPW
        q = u % QPW
        off = pl.multiple_of(l * B + b0 + q * G, G)
        pltpu.sync_copy(x_hbm.at[pl.ds(off, G)], idx[b])
        pltpu.make_async_copy(w_hbm.at[idx[b]], pairs[b], sem_g[b]).start()

    def drain_gather(b):
        pltpu.make_async_copy(w_hbm.at[pl.ds(0, G)], pairs[b], sem_g[b]).wait()

    def out_copy(u, b):
        l = u // QPW
        bb = wid * QPW + (u % QPW)
        return pltpu.make_async_copy(
            stage[b].at[:, :, pl.ds(0, G)], out5_hbm.at[l, :, bb], sem_o[b])

    def drain_out(b):
        pltpu.make_async_copy(stage[b].at[:, :, pl.ds(0, G)],
                              out5_hbm.at[0, :, 0], sem_o[b]).wait()

    stage_idx(0, 0)

    lane = lax.iota(jnp.int32, LANES)
    dblk16 = [(lane + k * LANES) >> 3 for k in range(D // LANES)]
    sub16 = [(lane + k * LANES) & 7 for k in range(D // LANES)]

    def unit_body(uu, carry):
        for b in range(2):
            u = uu * 2 + b
            b2 = 1 - b
            drain_gather(b)

            @pl.when(u + 1 < UNITS)
            def _prefetch():
                stage_idx(u + 1, b2)

            @pl.when(u >= 2)
            def _():
                drain_out(b)

            q = u % QPW

            def sel_body(r, carry2):
                r16 = jnp.full((LANES,), r, jnp.int32)
                for k in range(D // LANES):
                    v = pairs[b][r, pl.ds(k * LANES, LANES)]
                    plsc.store_scatter(stage[b], [dblk16[k], sub16[k], r16], v)
                return carry2

            lax.fori_loop(0, G, sel_body, 0)

            def acc_body(dblk, carry2):
                for sub in range(8):
                    for g in range(NGRP):
                        sl = pl.ds(g * LANES, LANES)
                        plsc.addupdate(acc_v.at[dblk, q, sub, sl],
                                       stage[b][dblk, sub, sl])
                return carry2

            lax.fori_loop(0, DBLK, acc_body, 0)

            out_copy(u, b).start()
        return carry

    lax.fori_loop(0, UNITS // 2, unit_body, 0)
    drain_out(0)
    drain_out(1)

    def scale_body(dblk, carry):
        for q in range(QPW):
            for sub in range(8):
                for g in range(NGRP):
                    sl = pl.ds(g * LANES, LANES)
                    iv = inv_v[pl.ds(q * G + g * LANES, LANES)]
                    acc_v[dblk, q, sub, sl] = acc_v[dblk, q, sub, sl] * iv
        return carry

    lax.fori_loop(0, DBLK, scale_body, 0)
    pltpu.sync_copy(
        acc_v.at[:, :, :, pl.ds(0, G)],
        ret4_hbm.at[:, pl.ds(pl.multiple_of(wid * QPW, QPW), QPW)])


_embed_kernel = functools.partial(
    pl.kernel,
    out_type=(jax.ShapeDtypeStruct((L, DBLK, NBLK, 8, G), jnp.float32),
              jax.ShapeDtypeStruct((DBLK, NBLK, 8, G), jnp.float32)),
    mesh=plsc.VectorSubcoreMesh(core_axis_name="c", subcore_axis_name="s"),
    compiler_params=pltpu.CompilerParams(use_tc_tiling_on_sc=False,
                                         needs_layout_passes=False),
    scratch_types=[
        pltpu.VMEM((G,), jnp.int32),
        pltpu.VMEM((G,), jnp.int32),
        pltpu.VMEM((G, 2 * D), jnp.float32),
        pltpu.VMEM((G, 2 * D), jnp.float32),
        pltpu.VMEM((DBLK, 8, PITCH), jnp.float32),
        pltpu.VMEM((DBLK, 8, PITCH), jnp.float32),
        pltpu.VMEM((DBLK, QPW, 8, PITCH), jnp.float32),
        pltpu.VMEM((BPW,), jnp.float32),
        pltpu.SemaphoreType.DMA,
        pltpu.SemaphoreType.DMA,
        pltpu.SemaphoreType.DMA,
        pltpu.SemaphoreType.DMA,
    ],
)(_embed_body)


def kernel(x, x_len, weight):
    x1d = x.T.reshape(B * L).astype(jnp.int32)
    xlen = x_len.reshape(B).astype(jnp.float32)
    wpad = _wprep(weight.T)
    out5, ret4 = _embed_kernel(x1d, xlen, wpad)
    raw = jnp.transpose(out5, (2, 4, 0, 1, 3)).reshape(B, L, D)
    ret = jnp.transpose(ret4, (1, 3, 0, 2)).reshape(B, D)
    return (ret, raw)


# speedup vs baseline: 2.3498x; 2.3498x over previous
import functools
import jax
import jax.numpy as jnp
from jax import lax
from jax.experimental import pallas as pl
from jax.experimental.pallas import tpu as pltpu
from jax.experimental.pallas import tpu_sc as plsc

NC = 2
NS = 16
NW = NC * NS
LANES = 16

B = 16384
L = 50
D = 64
V = 1000000

G = 128
BPW = B // NW
QPW = BPW // G
UNITS = L * QPW
NBLK = B // G
DBLK = D // 8
NGRP = G // LANES
PITCH = 137

WBLK = 4096
WGRID = (V + WBLK - 1) // WBLK


def _wprep_body(i_ref, o_ref):
    o_ref[:, pl.ds(0, D)] = i_ref[...].T


_wprep = functools.partial(
    pl.pallas_call,
    grid=(WGRID,),
    in_specs=[pl.BlockSpec((D, WBLK), lambda g: (0, g))],
    out_specs=pl.BlockSpec((WBLK, 2 * D), lambda g: (g, 0)),
    out_shape=jax.ShapeDtypeStruct((V, 2 * D), jnp.float32),
)(_wprep_body)


def _embed_body(x_hbm, xlen_hbm, w_hbm, out5_hbm, ret4_hbm,
                idx0, idx1, pairs0, pairs1, stage0, stage1,
                acc_v, inv_v,
                sem_g0, sem_g1, sem_o0, sem_o1):
    idx = (idx0, idx1)
    pairs = (pairs0, pairs1)
    stage = (stage0, stage1)
    sem_g = (sem_g0, sem_g1)
    sem_o = (sem_o0, sem_o1)

    wid = lax.axis_index("s") * NC + lax.axis_index("c")
    b0 = wid * BPW

    pltpu.sync_copy(xlen_hbm.at[pl.ds(pl.multiple_of(b0, BPW), BPW)], inv_v)
    for k in range(BPW // LANES):
        inv_v[pl.ds(k * LANES, LANES)] = 1.0 / inv_v[pl.ds(k * LANES, LANES)]

    zero = jnp.zeros((LANES,), jnp.float32)

    def zero_body(dblk, carry):
        for q in range(QPW):
            for sub in range(8):
                for g in range(NGRP):
                    acc_v[dblk, q, sub, pl.ds(g * LANES, LANES)] = zero
        return carry

    lax.fori_loop(0, DBLK, zero_body, 0)

    def stage_idx(u, b):
        l = u // QPW
        q = u % QPW
        off = pl.multiple_of(l * B + b0 + q * G, G)
        pltpu.sync_copy(x_hbm.at[pl.ds(off, G)], idx[b])
        pltpu.make_async_copy(w_hbm.at[idx[b]], pairs[b], sem_g[b]).start()

    def drain_gather(b):
        pltpu.make_async_copy(w_hbm.at[pl.ds(0, G)], pairs[b], sem_g[b]).wait()

    def out_copy(u, b):
        l = u // QPW
        bb = wid * QPW + (u % QPW)
        return pltpu.make_async_copy(
            stage[b].at[:, :, pl.ds(0, G)], out5_hbm.at[l, :, bb], sem_o[b])

    def drain_out(b):
        pltpu.make_async_copy(stage[b].at[:, :, pl.ds(0, G)],
                              out5_hbm.at[0, :, 0], sem_o[b]).wait()

    stage_idx(0, 0)

    lane = lax.iota(jnp.int32, LANES)
    dblk16 = [(lane + k * LANES) >> 3 for k in range(D // LANES)]
    sub16 = [(lane + k * LANES) & 7 for k in range(D // LANES)]

    def unit_body(uu, carry):
        for b in range(2):
            u = uu * 2 + b
            b2 = 1 - b
            drain_gather(b)

            @pl.when(u + 1 < UNITS)
            def _prefetch():
                stage_idx(u + 1, b2)

            @pl.when(u >= 2)
            def _():
                drain_out(b)

            q = u % QPW

            def sel_body(r4, carry2):
                for j in range(4):
                    r = r4 * 4 + j
                    r16 = jnp.full((LANES,), r, jnp.int32)
                    for k in range(D // LANES):
                        v = pairs[b][r, pl.ds(k * LANES, LANES)]
                        plsc.store_scatter(stage[b],
                                           [dblk16[k], sub16[k], r16], v)
                return carry2

            lax.fori_loop(0, G // 4, sel_body, 0)

            def acc_body(dblk, carry2):
                for sub in range(8):
                    for g in range(NGRP):
                        sl = pl.ds(g * LANES, LANES)
                        plsc.addupdate(acc_v.at[dblk, q, sub, sl],
                                       stage[b][dblk, sub, sl])
                return carry2

            lax.fori_loop(0, DBLK, acc_body, 0)

            out_copy(u, b).start()
        return carry

    lax.fori_loop(0, UNITS // 2, unit_body, 0)
    drain_out(0)
    drain_out(1)

    def scale_body(dblk, carry):
        for q in range(QPW):
            for sub in range(8):
                for g in range(NGRP):
                    sl = pl.ds(g * LANES, LANES)
                    iv = inv_v[pl.ds(q * G + g * LANES, LANES)]
                    acc_v[dblk, q, sub, sl] = acc_v[dblk, q, sub, sl] * iv
        return carry

    lax.fori_loop(0, DBLK, scale_body, 0)
    pltpu.sync_copy(
        acc_v.at[:, :, :, pl.ds(0, G)],
        ret4_hbm.at[:, pl.ds(pl.multiple_of(wid * QPW, QPW), QPW)])


_embed_kernel = functools.partial(
    pl.kernel,
    out_type=(jax.ShapeDtypeStruct((L, DBLK, NBLK, 8, G), jnp.float32),
              jax.ShapeDtypeStruct((DBLK, NBLK, 8, G), jnp.float32)),
    mesh=plsc.VectorSubcoreMesh(core_axis_name="c", subcore_axis_name="s"),
    compiler_params=pltpu.CompilerParams(use_tc_tiling_on_sc=False,
                                         needs_layout_passes=False),
    scratch_types=[
        pltpu.VMEM((G,), jnp.int32),
        pltpu.VMEM((G,), jnp.int32),
        pltpu.VMEM((G, 2 * D), jnp.float32),
        pltpu.VMEM((G, 2 * D), jnp.float32),
        pltpu.VMEM((DBLK, 8, PITCH), jnp.float32),
        pltpu.VMEM((DBLK, 8, PITCH), jnp.float32),
        pltpu.VMEM((DBLK, QPW, 8, PITCH), jnp.float32),
        pltpu.VMEM((BPW,), jnp.float32),
        pltpu.SemaphoreType.DMA,
        pltpu.SemaphoreType.DMA,
        pltpu.SemaphoreType.DMA,
        pltpu.SemaphoreType.DMA,
    ],
)(_embed_body)


def kernel(x, x_len, weight):
    x1d = x.T.reshape(B * L).astype(jnp.int32)
    xlen = x_len.reshape(B).astype(jnp.float32)
    wpad = _wprep(weight.T)
    out5, ret4 = _embed_kernel(x1d, xlen, wpad)
    raw = jnp.transpose(out5, (2, 4, 0, 1, 3)).reshape(B, L, D)
    ret = jnp.transpose(ret4, (1, 3, 0, 2)).reshape(B, D)
    return (ret, raw)


# speedup vs baseline: 2.8711x; 1.2218x over previous
import functools
import jax
import jax.numpy as jnp
from jax import lax
from jax.experimental import pallas as pl
from jax.experimental.pallas import tpu as pltpu
from jax.experimental.pallas import tpu_sc as plsc

NC = 2
NS = 16
NW = NC * NS
LANES = 16

B = 16384
L = 50
D = 64
V = 1000000

G = 128
BPW = B // NW
QPW = BPW // G
UNITS = L * QPW
NBLK = B // G
DBLK = D // 8
NGRP = G // LANES
PITCH = 137

WBLK = 4096
WGRID = (V + WBLK - 1) // WBLK


def _wprep_body(i_ref, o_ref):
    o_ref[:, pl.ds(0, D)] = i_ref[...].T


_wprep = functools.partial(
    pl.pallas_call,
    grid=(WGRID,),
    in_specs=[pl.BlockSpec((D, WBLK), lambda g: (0, g))],
    out_specs=pl.BlockSpec((WBLK, 2 * D), lambda g: (g, 0)),
    out_shape=jax.ShapeDtypeStruct((V, 2 * D), jnp.float32),
)(_wprep_body)


def _embed_body(x_hbm, xlen_hbm, w_hbm, out5_hbm, ret4_hbm,
                idx0, idx1, idx2, idx3, pairs0, pairs1, pairs2, pairs3,
                stage0, stage1, acc_d, inv_v,
                sem_g0, sem_g1, sem_g2, sem_g3, sem_o0, sem_o1):
    idx = (idx0, idx1, idx2, idx3)
    pairs = (pairs0, pairs1, pairs2, pairs3)
    stage = (stage0, stage1)
    sem_g = (sem_g0, sem_g1, sem_g2, sem_g3)
    sem_o = (sem_o0, sem_o1)

    wid = lax.axis_index("s") * NC + lax.axis_index("c")
    b0 = wid * BPW

    pltpu.sync_copy(xlen_hbm.at[pl.ds(pl.multiple_of(b0, BPW), BPW)], inv_v)
    for k in range(BPW // LANES):
        inv_v[pl.ds(k * LANES, LANES)] = 1.0 / inv_v[pl.ds(k * LANES, LANES)]

    zero = jnp.zeros((LANES,), jnp.float32)

    def zero_body(r, carry):
        for k in range(D // LANES):
            acc_d[r, pl.ds(k * LANES, LANES)] = zero
        return carry

    lax.fori_loop(0, BPW, zero_body, 0)

    def stage_idx(u, b):
        l = u // QPW
        q = u % QPW
        off = pl.multiple_of(l * B + b0 + q * G, G)
        pltpu.sync_copy(x_hbm.at[pl.ds(off, G)], idx[b])
        pltpu.make_async_copy(w_hbm.at[idx[b]], pairs[b], sem_g[b]).start()

    def drain_gather(b):
        pltpu.make_async_copy(w_hbm.at[pl.ds(0, G)], pairs[b], sem_g[b]).wait()

    def out_copy(u, b):
        l = u // QPW
        bb = wid * QPW + (u % QPW)
        return pltpu.make_async_copy(
            stage[b].at[:, :, pl.ds(0, G)], out5_hbm.at[l, :, bb], sem_o[b])

    def drain_out(b):
        pltpu.make_async_copy(stage[b].at[:, :, pl.ds(0, G)],
                              out5_hbm.at[0, :, 0], sem_o[b]).wait()

    stage_idx(0, 0)
    stage_idx(1, 1)
    stage_idx(2, 2)

    lane = lax.iota(jnp.int32, LANES)
    dblk16 = [(lane + k * LANES) >> 3 for k in range(D // LANES)]
    sub16 = [(lane + k * LANES) & 7 for k in range(D // LANES)]

    def unit_body(uu, carry):
        for b in range(4):
            u = uu * 4 + b
            sb = b % 2
            drain_gather(b)

            @pl.when(u + 3 < UNITS)
            def _prefetch():
                stage_idx(u + 3, (b + 3) % 4)

            @pl.when(u >= 2)
            def _():
                drain_out(sb)

            q = u % QPW
            arow = pl.multiple_of(q * G, G)

            def sel_body(r4, carry2):
                for j in range(4):
                    r = r4 * 4 + j
                    r16 = jnp.full((LANES,), r, jnp.int32)
                    for k in range(D // LANES):
                        sl = pl.ds(k * LANES, LANES)
                        v = pairs[b][r, sl]
                        plsc.store_scatter(stage[sb],
                                           [dblk16[k], sub16[k], r16], v)
                        plsc.addupdate(acc_d.at[arow + r, sl], v)
                return carry2

            lax.fori_loop(0, G // 4, sel_body, 0)

            out_copy(u, sb).start()
        return carry

    lax.fori_loop(0, UNITS // 4, unit_body, 0)
    drain_out(0)
    drain_out(1)

    for q in range(QPW):
        def ret_body(r, carry):
            r16 = jnp.full((LANES,), r, jnp.int32)
            for k in range(D // LANES):
                v = acc_d[q * G + r, pl.ds(k * LANES, LANES)]
                plsc.store_scatter(stage[0], [dblk16[k], sub16[k], r16], v)
            return carry

        lax.fori_loop(0, G, ret_body, 0)

        def ret_scale(dblk, carry):
            for sub in range(8):
                for g in range(NGRP):
                    sl = pl.ds(g * LANES, LANES)
                    iv = inv_v[pl.ds(q * G + g * LANES, LANES)]
                    stage[0][dblk, sub, sl] = stage[0][dblk, sub, sl] * iv
            return carry

        lax.fori_loop(0, DBLK, ret_scale, 0)
        pltpu.sync_copy(stage[0].at[:, :, pl.ds(0, G)],
                        ret4_hbm.at[:, wid * QPW + q])


_embed_kernel = functools.partial(
    pl.kernel,
    out_type=(jax.ShapeDtypeStruct((L, DBLK, NBLK, 8, G), jnp.float32),
              jax.ShapeDtypeStruct((DBLK, NBLK, 8, G), jnp.float32)),
    mesh=plsc.VectorSubcoreMesh(core_axis_name="c", subcore_axis_name="s"),
    compiler_params=pltpu.CompilerParams(use_tc_tiling_on_sc=False,
                                         needs_layout_passes=False),
    scratch_types=(
        [pltpu.VMEM((G,), jnp.int32) for _ in range(4)]
        + [pltpu.VMEM((G, 2 * D), jnp.float32) for _ in range(4)]
        + [pltpu.VMEM((DBLK, 8, PITCH), jnp.float32) for _ in range(2)]
        + [pltpu.VMEM((BPW, D), jnp.float32),
           pltpu.VMEM((BPW,), jnp.float32)]
        + [pltpu.SemaphoreType.DMA for _ in range(6)]
    ),
)(_embed_body)


def kernel(x, x_len, weight):
    x1d = x.T.reshape(B * L).astype(jnp.int32)
    xlen = x_len.reshape(B).astype(jnp.float32)
    wpad = _wprep(weight.T)
    out5, ret4 = _embed_kernel(x1d, xlen, wpad)
    raw = jnp.transpose(out5, (2, 4, 0, 1, 3)).reshape(B, L, D)
    ret = jnp.transpose(ret4, (1, 3, 0, 2)).reshape(B, D)
    return (ret, raw)


# speedup vs baseline: 2.8891x; 1.0063x over previous
import functools
import jax
import jax.numpy as jnp
from jax import lax
from jax.experimental import pallas as pl
from jax.experimental.pallas import tpu as pltpu
from jax.experimental.pallas import tpu_sc as plsc

NC = 2
NS = 16
NW = NC * NS
LANES = 16

B = 16384
L = 50
D = 64
V = 1000000

G = 128
BPW = B // NW
QPW = BPW // G
UNITS = L * QPW
NBLK = B // G
DBLK = D // 8
NGRP = G // LANES
PITCH = 137

WBLK = 4096
WGRID = (V + WBLK - 1) // WBLK


def _wprep_body(i_ref, o_ref):
    o_ref[:, pl.ds(0, D)] = i_ref[...].T


_wprep = functools.partial(
    pl.pallas_call,
    grid=(WGRID,),
    in_specs=[pl.BlockSpec((D, WBLK), lambda g: (0, g))],
    out_specs=pl.BlockSpec((WBLK, 2 * D), lambda g: (g, 0)),
    out_shape=jax.ShapeDtypeStruct((V, 2 * D), jnp.float32),
)(_wprep_body)


def _embed_body(x_hbm, xlen_hbm, w_hbm, out5_hbm, ret4_hbm,
                idx0, idx1, idx2, idx3, pairs0, pairs1, pairs2, pairs3,
                stage0, stage1, acc_d, inv_v,
                sem_g0, sem_g1, sem_g2, sem_g3, sem_o0, sem_o1):
    idx = (idx0, idx1, idx2, idx3)
    pairs = (pairs0, pairs1, pairs2, pairs3)
    stage = (stage0, stage1)
    sem_g = (sem_g0, sem_g1, sem_g2, sem_g3)
    sem_o = (sem_o0, sem_o1)

    wid = lax.axis_index("s") * NC + lax.axis_index("c")
    b0 = wid * BPW

    pltpu.sync_copy(xlen_hbm.at[pl.ds(pl.multiple_of(b0, BPW), BPW)], inv_v)
    for k in range(BPW // LANES):
        inv_v[pl.ds(k * LANES, LANES)] = 1.0 / inv_v[pl.ds(k * LANES, LANES)]

    zero = jnp.zeros((LANES,), jnp.float32)

    def zero_body(r, carry):
        for k in range(D // LANES):
            acc_d[r, pl.ds(k * LANES, LANES)] = zero
        return carry

    lax.fori_loop(0, BPW, zero_body, 0)

    def stage_idx(u, b):
        l = u // QPW
        q = u % QPW
        off = pl.multiple_of(l * B + b0 + q * G, G)
        pltpu.sync_copy(x_hbm.at[pl.ds(off, G)], idx[b])
        for j in range(2):
            pltpu.make_async_copy(
                w_hbm.at[idx[b].at[pl.ds(j * (G // 2), G // 2)]],
                pairs[b].at[pl.ds(j * (G // 2), G // 2)],
                sem_g[b]).start()

    def drain_gather(b):
        pltpu.make_async_copy(w_hbm.at[pl.ds(0, G)], pairs[b], sem_g[b]).wait()

    def out_copy(u, b):
        l = u // QPW
        bb = wid * QPW + (u % QPW)
        return pltpu.make_async_copy(
            stage[b].at[:, :, pl.ds(0, G)], out5_hbm.at[l, :, bb], sem_o[b])

    def drain_out(b):
        pltpu.make_async_copy(stage[b].at[:, :, pl.ds(0, G)],
                              out5_hbm.at[0, :, 0], sem_o[b]).wait()

    stage_idx(0, 0)
    stage_idx(1, 1)
    stage_idx(2, 2)

    lane = lax.iota(jnp.int32, LANES)
    dblk16 = [(lane + k * LANES) >> 3 for k in range(D // LANES)]
    sub16 = [(lane + k * LANES) & 7 for k in range(D // LANES)]

    def unit_body(uu, carry):
        for b in range(4):
            u = uu * 4 + b
            sb = b % 2
            drain_gather(b)

            @pl.when(u + 3 < UNITS)
            def _prefetch():
                stage_idx(u + 3, (b + 3) % 4)

            @pl.when(u >= 2)
            def _():
                drain_out(sb)

            q = u % QPW
            arow = pl.multiple_of(q * G, G)

            def sel_body(r4, carry2):
                for j in range(4):
                    r = r4 * 4 + j
                    r16 = jnp.full((LANES,), r, jnp.int32)
                    for k in range(D // LANES):
                        sl = pl.ds(k * LANES, LANES)
                        v = pairs[b][r, sl]
                        plsc.store_scatter(stage[sb],
                                           [dblk16[k], sub16[k], r16], v)
                        plsc.addupdate(acc_d.at[arow + r, sl], v)
                return carry2

            lax.fori_loop(0, G // 4, sel_body, 0)

            out_copy(u, sb).start()
        return carry

    lax.fori_loop(0, UNITS // 4, unit_body, 0)
    drain_out(0)
    drain_out(1)

    for q in range(QPW):
        def ret_body(r, carry):
            r16 = jnp.full((LANES,), r, jnp.int32)
            for k in range(D // LANES):
                v = acc_d[q * G + r, pl.ds(k * LANES, LANES)]
                plsc.store_scatter(stage[0], [dblk16[k], sub16[k], r16], v)
            return carry

        lax.fori_loop(0, G, ret_body, 0)

        def ret_scale(dblk, carry):
            for sub in range(8):
                for g in range(NGRP):
                    sl = pl.ds(g * LANES, LANES)
                    iv = inv_v[pl.ds(q * G + g * LANES, LANES)]
                    stage[0][dblk, sub, sl] = stage[0][dblk, sub, sl] * iv
            return carry

        lax.fori_loop(0, DBLK, ret_scale, 0)
        pltpu.sync_copy(stage[0].at[:, :, pl.ds(0, G)],
                        ret4_hbm.at[:, wid * QPW + q])


_embed_kernel = functools.partial(
    pl.kernel,
    out_type=(jax.ShapeDtypeStruct((L, DBLK, NBLK, 8, G), jnp.float32),
              jax.ShapeDtypeStruct((DBLK, NBLK, 8, G), jnp.float32)),
    mesh=plsc.VectorSubcoreMesh(core_axis_name="c", subcore_axis_name="s"),
    compiler_params=pltpu.CompilerParams(use_tc_tiling_on_sc=False,
                                         needs_layout_passes=False),
    scratch_types=(
        [pltpu.VMEM((G,), jnp.int32) for _ in range(4)]
        + [pltpu.VMEM((G, 2 * D), jnp.float32) for _ in range(4)]
        + [pltpu.VMEM((DBLK, 8, PITCH), jnp.float32) for _ in range(2)]
        + [pltpu.VMEM((BPW, D), jnp.float32),
           pltpu.VMEM((BPW,), jnp.float32)]
        + [pltpu.SemaphoreType.DMA for _ in range(6)]
    ),
)(_embed_body)


def kernel(x, x_len, weight):
    x1d = x.T.reshape(B * L).astype(jnp.int32)
    xlen = x_len.reshape(B).astype(jnp.float32)
    wpad = _wprep(weight.T)
    out5, ret4 = _embed_kernel(x1d, xlen, wpad)
    raw = jnp.transpose(out5, (2, 4, 0, 1, 3)).reshape(B, L, D)
    ret = jnp.transpose(ret4, (1, 3, 0, 2)).reshape(B, D)
    return (ret, raw)
